# Initial kernel scaffold; baseline (speedup 1.0000x reference)
import jax
import jax.numpy as jnp
from jax.experimental import pallas as pl


def kernel(x, edge_index, edge_attr, batch, params):
    raise NotImplementedError("write your pallas kernel here")


# speedup vs baseline: 2.1207x; 2.1207x over previous
import functools

import jax
import jax.numpy as jnp
from jax import lax
from jax.experimental import pallas as pl
from jax.experimental.pallas import tpu as pltpu
from jax.experimental.pallas import tpu_sc as plsc

F32 = jnp.float32
BF16 = jnp.bfloat16

N_NODES = 10000
N_EDGES = 160000
NUM_GRAPHS = 64
D_NODE = 128

NC, NS = 2, 16
NW = NC * NS
CHUNK = 128
N_CHUNKS = N_EDGES // CHUNK
CHUNKS_PER_SC = N_CHUNKS // NC
N_PAD = 10240
ROWS_PER_TILE = N_PAD // NS

BLK_E = 2000
BLK_N = 2000

_SC_MESH = dict(core_axis_name="c", subcore_axis_name="s")


def _sc_gather2(x, row, col):
    mesh = plsc.VectorSubcoreMesh(**_SC_MESH)

    @functools.partial(
        pl.kernel,
        out_type=(jax.ShapeDtypeStruct((N_EDGES, D_NODE), F32),
                  jax.ShapeDtypeStruct((N_EDGES, D_NODE), F32)),
        mesh=mesh,
        scratch_types=[
            pltpu.VMEM((CHUNK,), jnp.int32),
            pltpu.VMEM((CHUNK, D_NODE), F32),
            pltpu.VMEM((CHUNK,), jnp.int32),
            pltpu.VMEM((CHUNK, D_NODE), F32),
            pltpu.SemaphoreType.DMA,
            pltpu.SemaphoreType.DMA,
        ],
    )
    def k(x_hbm, row_hbm, col_hbm, xr_hbm, xc_hbm, ridx, rbuf, cidx, cbuf,
          sem_r, sem_c):
        wid = lax.axis_index("s") * NC + lax.axis_index("c")

        def do_chunk(base):
            pltpu.sync_copy(row_hbm.at[pl.ds(base, CHUNK)], ridx)
            pltpu.async_copy(x_hbm.at[ridx], rbuf, sem_r).wait()
            pltpu.sync_copy(rbuf, xr_hbm.at[pl.ds(base, CHUNK)])
            pltpu.sync_copy(col_hbm.at[pl.ds(base, CHUNK)], cidx)
            pltpu.async_copy(x_hbm.at[cidx], cbuf, sem_c).wait()
            pltpu.sync_copy(cbuf, xc_hbm.at[pl.ds(base, CHUNK)])

        def body(i, carry):
            do_chunk((wid + i * NW) * CHUNK)
            return carry

        lax.fori_loop(0, N_CHUNKS // NW, body, 0)

        @pl.when(wid < N_CHUNKS % NW)
        def _():
            do_chunk(((N_CHUNKS // NW) * NW + wid) * CHUNK)

    return k(x, row, col)


def _sc_segment_sum(m, col, zrows):
    mesh = plsc.VectorSubcoreMesh(**_SC_MESH)

    @functools.partial(
        pl.kernel,
        out_type=jax.ShapeDtypeStruct((NC, N_PAD, D_NODE), F32),
        mesh=mesh,
        scratch_types=[
            pltpu.VMEM((CHUNK,), jnp.int32),
            pltpu.VMEM((CHUNK, D_NODE), F32),
            pltpu.VMEM_SHARED((N_PAD, D_NODE), F32),
        ],
    )
    def k(m_hbm, col_hbm, z_hbm, out, idx, buf, acc):
        cid = lax.axis_index("c")
        sid = lax.axis_index("s")
        my_rows = acc.at[pl.ds(sid * ROWS_PER_TILE, ROWS_PER_TILE)]
        pltpu.sync_copy(z_hbm, my_rows)
        plsc.subcore_barrier()

        def do_chunk(c):
            base = c * CHUNK
            pltpu.sync_copy(col_hbm.at[pl.ds(base, CHUNK)], idx)
            pltpu.sync_copy(m_hbm.at[pl.ds(base, CHUNK)], buf)
            pltpu.sync_copy(buf, acc.at[idx], add=True)

        def body(i, carry):
            do_chunk(cid * CHUNKS_PER_SC + sid + i * NS)
            return carry

        lax.fori_loop(0, CHUNKS_PER_SC // NS, body, 0)

        @pl.when(sid < CHUNKS_PER_SC % NS)
        def _():
            do_chunk(cid * CHUNKS_PER_SC + (CHUNKS_PER_SC // NS) * NS + sid)

        plsc.subcore_barrier()
        pltpu.sync_copy(
            my_rows, out.at[cid, pl.ds(sid * ROWS_PER_TILE, ROWS_PER_TILE)])

    return k(m, col, zrows)


def _bn_x(x, gamma, beta):
    def body(x_ref, g_ref, b_ref, o_ref):
        xx = x_ref[...]
        mu = jnp.mean(xx, axis=0, keepdims=True)
        var = jnp.mean(xx * xx, axis=0, keepdims=True) - mu * mu
        o_ref[...] = (xx - mu) * lax.rsqrt(var + 1e-5) * g_ref[...] + b_ref[...]

    return pl.pallas_call(
        body, out_shape=jax.ShapeDtypeStruct(x.shape, F32),
    )(x, gamma.reshape(1, -1), beta.reshape(1, -1))


def _bn_e_stats(e2):
    def body(e_ref, s_ref, q_ref):
        e = e_ref[...]
        s_ref[...] = jnp.sum(e, axis=0, keepdims=True)
        q_ref[...] = jnp.sum(e * e, axis=0, keepdims=True)

    return pl.pallas_call(
        body,
        out_shape=(jax.ShapeDtypeStruct((1, 128), F32),
                   jax.ShapeDtypeStruct((1, 128), F32)),
    )(e2)


def _pool_precompute(batch2d):
    nblk = N_NODES // BLK_N

    def body(b_ref, ot_ref, g_ref):
        i = pl.program_id(0)
        b = b_ref[...]
        gid = lax.broadcasted_iota(jnp.int32, (BLK_N, NUM_GRAPHS), 1)
        ot = (gid == b).astype(F32)
        ot_ref[...] = ot
        ones = jnp.ones((BLK_N, D_NODE), F32)
        gs = lax.dot_general(ot, ones, (((0,), (0,)), ((), ())),
                             preferred_element_type=F32)

        @pl.when(i == 0)
        def _():
            g_ref[...] = gs

        @pl.when(i > 0)
        def _():
            g_ref[...] += gs

    return pl.pallas_call(
        body,
        grid=(nblk,),
        in_specs=[pl.BlockSpec((BLK_N, 1), lambda i: (i, 0))],
        out_specs=(pl.BlockSpec((BLK_N, NUM_GRAPHS), lambda i: (i, 0)),
                   pl.BlockSpec((NUM_GRAPHS, D_NODE), lambda i: (0, 0))),
        out_shape=(jax.ShapeDtypeStruct((N_NODES, NUM_GRAPHS), F32),
                   jax.ShapeDtypeStruct((NUM_GRAPHS, D_NODE), F32)),
    )(batch2d)


def _edge_layer(xr, xc, e, wer, wec, wee, be, wn1x, wn1e, bn1, store_e):
    d_in = e.shape[1]
    e_out = wee.shape[1]
    nblk = N_EDGES // BLK_E

    def body(xr_ref, xc_ref, e_ref, wer_ref, wec_ref, wee_ref, be_ref,
             wn1x_ref, wn1e_ref, bn1_ref, *out_refs):
        xr_b = xr_ref[...].astype(BF16)
        xc_b = xc_ref[...].astype(BF16)
        e_b = e_ref[...].astype(BF16)
        acc = jnp.dot(xr_b, wer_ref[...].astype(BF16),
                      preferred_element_type=F32)
        acc += jnp.dot(xc_b, wec_ref[...].astype(BF16),
                       preferred_element_type=F32)
        acc += jnp.dot(e_b, wee_ref[...].astype(BF16),
                       preferred_element_type=F32)
        enew = jnp.maximum(acc + be_ref[...], 0.0)
        m = jnp.dot(xr_b, wn1x_ref[...].astype(BF16),
                    preferred_element_type=F32)
        m += jnp.dot(enew.astype(BF16), wn1e_ref[...].astype(BF16),
                     preferred_element_type=F32)
        m = jnp.maximum(m + bn1_ref[...], 0.0)
        if store_e:
            out_refs[0][...] = enew
            out_refs[1][...] = m
        else:
            out_refs[0][...] = m

    full = lambda shape: pl.BlockSpec(shape, lambda i: (0, 0))
    out_specs = [pl.BlockSpec((BLK_E, e_out), lambda i: (i, 0)),
                 pl.BlockSpec((BLK_E, D_NODE), lambda i: (i, 0))]
    out_shape = [jax.ShapeDtypeStruct((N_EDGES, e_out), F32),
                 jax.ShapeDtypeStruct((N_EDGES, D_NODE), F32)]
    if not store_e:
        out_specs, out_shape = out_specs[1:], out_shape[1:]

    res = pl.pallas_call(
        body,
        grid=(nblk,),
        in_specs=[
            pl.BlockSpec((BLK_E, D_NODE), lambda i: (i, 0)),
            pl.BlockSpec((BLK_E, D_NODE), lambda i: (i, 0)),
            pl.BlockSpec((BLK_E, d_in), lambda i: (i, 0)),
            full((D_NODE, e_out)),
            full((D_NODE, e_out)),
            full((d_in, e_out)),
            full((1, e_out)),
            full((D_NODE, D_NODE)),
            full((e_out, D_NODE)),
            full((1, D_NODE)),
        ],
        out_specs=tuple(out_specs),
        out_shape=tuple(out_shape),
    )(xr, xc, e, wer, wec, wee, be.reshape(1, -1), wn1x, wn1e,
      bn1.reshape(1, -1))
    return res if store_e else (None, res[0])


def _node_layer(x, msum, cnts, ot, w2x, w2a, b2):
    nblk = N_NODES // BLK_N

    def body(x_ref, ms_ref, cnt_ref, ot_ref,
             w2x_ref, w2a_ref, b2_ref, xn_ref, ps_ref):
        i = pl.program_id(0)
        ms = ms_ref[0] + ms_ref[1]
        cnt = cnt_ref[0] + cnt_ref[1]
        inv = 1.0 / jnp.maximum(cnt[:, 0:1], 1.0)
        agg = ms * inv
        xn = jnp.dot(x_ref[...].astype(BF16), w2x_ref[...].astype(BF16),
                     preferred_element_type=F32)
        xn += jnp.dot(agg.astype(BF16), w2a_ref[...].astype(BF16),
                      preferred_element_type=F32)
        xn += b2_ref[...]
        xn_ref[...] = xn
        pp = lax.dot_general(ot_ref[...], xn, (((0,), (0,)), ((), ())),
                             preferred_element_type=F32)

        @pl.when(i == 0)
        def _():
            ps_ref[...] = pp

        @pl.when(i > 0)
        def _():
            ps_ref[...] += pp

    full = lambda shape: pl.BlockSpec(shape, lambda i: (0, 0))
    return pl.pallas_call(
        body,
        grid=(nblk,),
        in_specs=[
            pl.BlockSpec((BLK_N, D_NODE), lambda i: (i, 0)),
            pl.BlockSpec((NC, BLK_N, D_NODE), lambda i: (0, i, 0)),
            pl.BlockSpec((NC, BLK_N, D_NODE), lambda i: (0, i, 0)),
            pl.BlockSpec((BLK_N, NUM_GRAPHS), lambda i: (i, 0)),
            full((D_NODE, D_NODE)),
            full((D_NODE, D_NODE)),
            full((1, D_NODE)),
        ],
        out_specs=(pl.BlockSpec((BLK_N, D_NODE), lambda i: (i, 0)),
                   pl.BlockSpec((NUM_GRAPHS, D_NODE), lambda i: (0, 0))),
        out_shape=(jax.ShapeDtypeStruct((N_NODES, D_NODE), F32),
                   jax.ShapeDtypeStruct((NUM_GRAPHS, D_NODE), F32)),
    )(x, msum, cnts, ot, w2x, w2a, b2.reshape(1, -1))


def _glob_layer(psum, gcnt, u, wg_u, wg_p, bg):
    g_out = wg_p.shape[1]
    has_u = u is not None

    def body(*refs):
        if has_u:
            ps_ref, g_ref, u_ref, wgu_ref, wgp_ref, bg_ref, o_ref = refs
        else:
            ps_ref, g_ref, wgp_ref, bg_ref, o_ref = refs
        pooled = ps_ref[...] / jnp.maximum(g_ref[...], 1.0)
        acc = jnp.dot(pooled, wgp_ref[...], preferred_element_type=F32)
        if has_u:
            acc += jnp.dot(u_ref[...], wgu_ref[...], preferred_element_type=F32)
        o_ref[...] = acc + bg_ref[...]

    args = (psum, gcnt) + ((u, wg_u) if has_u else ()) + (wg_p, bg.reshape(1, -1))
    return pl.pallas_call(
        body, out_shape=jax.ShapeDtypeStruct((NUM_GRAPHS, g_out), F32),
    )(*args)


def _head(u, w1, b1, w2, b2):
    def body(u_ref, w1_ref, b1_ref, w2_ref, b2_ref, o_ref):
        y = jnp.dot(u_ref[...], w1_ref[...], preferred_element_type=F32)
        y += b1_ref[...]
        y = jnp.where(y > 0, y, jnp.exp(y) - 1.0)
        o_ref[...] = jnp.dot(y, w2_ref[...],
                             preferred_element_type=F32) + b2_ref[...]

    return pl.pallas_call(
        body, out_shape=jax.ShapeDtypeStruct((NUM_GRAPHS, w2.shape[1]), F32),
    )(u, w1, b1.reshape(1, -1), w2, b2.reshape(1, -1))


def kernel(x, edge_index, edge_attr, batch, params):
    row = edge_index[0].astype(jnp.int32)
    col = edge_index[1].astype(jnp.int32)
    batch2d = batch.astype(jnp.int32).reshape(N_NODES, 1)

    zrows = jnp.zeros((ROWS_PER_TILE, D_NODE), F32)

    xcur = _bn_x(x, params["bn_node"]["gamma"], params["bn_node"]["beta"])

    s128, q128 = _bn_e_stats(edge_attr.reshape(N_EDGES // 8, 128))
    s16 = s128.reshape(8, 16).sum(0)
    q16 = q128.reshape(8, 16).sum(0)
    mu_e = s16 / N_EDGES
    var_e = q16 / N_EDGES - mu_e * mu_e
    scale_e = params["bn_edge"]["gamma"] * lax.rsqrt(var_e + 1e-5)
    shift_e = params["bn_edge"]["beta"] - mu_e * scale_e

    cnts = _sc_segment_sum(jnp.ones((N_EDGES, D_NODE), F32), col, zrows)
    ot, gcnt = _pool_precompute(batch2d)

    ecur = edge_attr
    u = None
    for li in range(6):
        p = params["meta%d" % (li + 1)]
        we = p["edge"]["w"]
        be = p["edge"]["b"]
        wer, wec, wee = we[:D_NODE], we[D_NODE:2 * D_NODE], we[2 * D_NODE:]
        if li == 0:
            wee = scale_e[:, None] * wee
            be = be + shift_e @ we[2 * D_NODE:]
        wn1 = p["node1"]["w"]
        wn1x, wn1e = wn1[:D_NODE], wn1[D_NODE:]
        xr, xc = _sc_gather2(xcur, row, col)
        enew, m = _edge_layer(xr, xc, ecur, wer, wec, wee, be,
                              wn1x, wn1e, p["node1"]["b"], store_e=(li < 5))
        msum = _sc_segment_sum(m, col, zrows)
        wn2 = p["node2"]["w"]
        xcur, psum = _node_layer(xcur, msum, cnts, ot,
                                 wn2[:D_NODE], wn2[D_NODE:], p["node2"]["b"])
        wg = p["glob"]["w"]
        if u is None:
            u = _glob_layer(psum, gcnt, None, None, wg, p["glob"]["b"])
        else:
            u_in = wg.shape[0] - D_NODE
            u = _glob_layer(psum, gcnt, u, wg[:u_in], wg[u_in:],
                            p["glob"]["b"])
        ecur = enew

    out2 = _head(u, params["lin1"]["w"], params["lin1"]["b"],
                 params["lin2"]["w"], params["lin2"]["b"])
    return (u, out2)


# speedup vs baseline: 2.7371x; 1.2906x over previous
import functools

import jax
import jax.numpy as jnp
from jax import lax
from jax.experimental import pallas as pl
from jax.experimental.pallas import tpu as pltpu
from jax.experimental.pallas import tpu_sc as plsc

F32 = jnp.float32
BF16 = jnp.bfloat16

N_NODES = 10000
N_EDGES = 160000
NUM_GRAPHS = 64
D_NODE = 128

NC, NS = 2, 16
NW = NC * NS
CHUNK = 128
N_CHUNKS = N_EDGES // CHUNK
CHUNKS_PER_SC = N_CHUNKS // NC
N_PAD = 10112
ROWS_PER_TILE = N_PAD // NS

BLK_E = 2000
BLK_N = 2000

_SC_MESH = dict(core_axis_name="c", subcore_axis_name="s")


def _sc_gather2(x, row, col):
    G = 3
    mesh = plsc.VectorSubcoreMesh(**_SC_MESH)

    @functools.partial(
        pl.kernel,
        out_type=(jax.ShapeDtypeStruct((N_EDGES, D_NODE), F32),
                  jax.ShapeDtypeStruct((N_EDGES, D_NODE), F32)),
        mesh=mesh,
        scratch_types=[
            pltpu.VMEM((G * CHUNK,), jnp.int32),
            pltpu.VMEM((G * CHUNK, D_NODE), F32),
            pltpu.VMEM((G * CHUNK,), jnp.int32),
            pltpu.VMEM((G * CHUNK, D_NODE), F32),
            pltpu.SemaphoreType.DMA,
            pltpu.SemaphoreType.DMA,
        ],
    )
    def k(x_hbm, row_hbm, col_hbm, xr_hbm, xc_hbm, ridx, rbuf, cidx, cbuf,
          sem_r, sem_c):
        wid = lax.axis_index("s") * NC + lax.axis_index("c")
        start = 39 * wid + jnp.minimum(wid, N_CHUNKS % NW)

        def group(i, carry):
            base = (start + i * G) * CHUNK
            pltpu.sync_copy(row_hbm.at[pl.ds(base, G * CHUNK)], ridx)
            pltpu.sync_copy(col_hbm.at[pl.ds(base, G * CHUNK)], cidx)
            ds = []
            for j in range(G):
                sl = pl.ds(j * CHUNK, CHUNK)
                ds.append(pltpu.async_copy(
                    x_hbm.at[ridx.at[sl]], rbuf.at[sl], sem_r))
                ds.append(pltpu.async_copy(
                    x_hbm.at[cidx.at[sl]], cbuf.at[sl], sem_c))
            for d in ds:
                d.wait()
            pltpu.sync_copy(rbuf, xr_hbm.at[pl.ds(base, G * CHUNK)])
            pltpu.sync_copy(cbuf, xc_hbm.at[pl.ds(base, G * CHUNK)])
            return carry

        lax.fori_loop(0, 39 // G, group, 0)

        @pl.when(wid < N_CHUNKS % NW)
        def _():
            base = (start + 39) * CHUNK
            one = pl.ds(0, CHUNK)
            pltpu.sync_copy(row_hbm.at[pl.ds(base, CHUNK)], ridx.at[one])
            pltpu.sync_copy(col_hbm.at[pl.ds(base, CHUNK)], cidx.at[one])
            dr = pltpu.async_copy(x_hbm.at[ridx.at[one]], rbuf.at[one], sem_r)
            dc = pltpu.async_copy(x_hbm.at[cidx.at[one]], cbuf.at[one], sem_c)
            dr.wait()
            dc.wait()
            pltpu.sync_copy(rbuf.at[one], xr_hbm.at[pl.ds(base, CHUNK)])
            pltpu.sync_copy(cbuf.at[one], xc_hbm.at[pl.ds(base, CHUNK)])

    return k(x, row, col)


def _sc_segment_sum(m, col, zrows):
    G = 3
    mesh = plsc.VectorSubcoreMesh(**_SC_MESH)

    @functools.partial(
        pl.kernel,
        out_type=jax.ShapeDtypeStruct((NC, N_PAD, D_NODE), F32),
        mesh=mesh,
        scratch_types=[
            pltpu.VMEM((CHUNK,), jnp.int32),
            pltpu.VMEM((CHUNK,), jnp.int32),
            pltpu.VMEM((CHUNK,), jnp.int32),
            pltpu.VMEM((G * CHUNK, D_NODE), F32),
            pltpu.VMEM_SHARED((N_PAD, D_NODE), F32),
            pltpu.SemaphoreType.DMA,
            pltpu.SemaphoreType.DMA,
        ],
    )
    def k(m_hbm, col_hbm, z_hbm, out, idx0, idx1, idx2, buf, acc, sem_i,
          sem_s):
        cid = lax.axis_index("c")
        sid = lax.axis_index("s")
        my_rows = acc.at[pl.ds(sid * ROWS_PER_TILE, ROWS_PER_TILE)]
        pltpu.sync_copy(z_hbm, my_rows)
        plsc.subcore_barrier()

        idxs = [idx0, idx1, idx2]
        start = cid * CHUNKS_PER_SC + 39 * sid + jnp.minimum(
            sid, CHUNKS_PER_SC % NS)

        def group(i, carry):
            c0 = start + i * G
            base = c0 * CHUNK
            dm = pltpu.async_copy(m_hbm.at[pl.ds(base, G * CHUNK)], buf,
                                  sem_s)
            di = [pltpu.async_copy(
                col_hbm.at[pl.ds(base + j * CHUNK, CHUNK)], idxs[j], sem_i)
                for j in range(G)]
            dm.wait()
            for d in di:
                d.wait()
            ds = [pltpu.async_copy(
                buf.at[pl.ds(j * CHUNK, CHUNK)], acc.at[idxs[j]], sem_s,
                add=True) for j in range(G)]
            for d in ds:
                d.wait()
            return carry

        lax.fori_loop(0, 39 // G, group, 0)

        @pl.when(sid < CHUNKS_PER_SC % NS)
        def _():
            base = (start + 39) * CHUNK
            pltpu.sync_copy(col_hbm.at[pl.ds(base, CHUNK)], idx0)
            pltpu.sync_copy(m_hbm.at[pl.ds(base, CHUNK)],
                            buf.at[pl.ds(0, CHUNK)])
            pltpu.sync_copy(buf.at[pl.ds(0, CHUNK)], acc.at[idx0], add=True)

        plsc.subcore_barrier()
        pltpu.sync_copy(
            my_rows, out.at[cid, pl.ds(sid * ROWS_PER_TILE, ROWS_PER_TILE)])

    return k(m, col, zrows)


def _bn_x(x, gamma, beta):
    def body(x_ref, g_ref, b_ref, o_ref):
        xx = x_ref[...]
        mu = jnp.mean(xx, axis=0, keepdims=True)
        var = jnp.mean(xx * xx, axis=0, keepdims=True) - mu * mu
        o_ref[...] = (xx - mu) * lax.rsqrt(var + 1e-5) * g_ref[...] + b_ref[...]

    return pl.pallas_call(
        body, out_shape=jax.ShapeDtypeStruct(x.shape, F32),
    )(x, gamma.reshape(1, -1), beta.reshape(1, -1))


def _bn_e_stats(e2):
    def body(e_ref, s_ref, q_ref):
        e = e_ref[...]
        s_ref[...] = jnp.sum(e, axis=0, keepdims=True)
        q_ref[...] = jnp.sum(e * e, axis=0, keepdims=True)

    return pl.pallas_call(
        body,
        out_shape=(jax.ShapeDtypeStruct((1, 128), F32),
                   jax.ShapeDtypeStruct((1, 128), F32)),
    )(e2)


def _pool_precompute(batch2d):
    nblk = N_NODES // BLK_N

    def body(b_ref, ot_ref, g_ref):
        i = pl.program_id(0)
        b = b_ref[...]
        gid = lax.broadcasted_iota(jnp.int32, (BLK_N, NUM_GRAPHS), 1)
        ot = (gid == b).astype(F32)
        ot_ref[...] = ot
        ones = jnp.ones((BLK_N, D_NODE), F32)
        gs = lax.dot_general(ot, ones, (((0,), (0,)), ((), ())),
                             preferred_element_type=F32)

        @pl.when(i == 0)
        def _():
            g_ref[...] = gs

        @pl.when(i > 0)
        def _():
            g_ref[...] += gs

    return pl.pallas_call(
        body,
        grid=(nblk,),
        in_specs=[pl.BlockSpec((BLK_N, 1), lambda i: (i, 0))],
        out_specs=(pl.BlockSpec((BLK_N, NUM_GRAPHS), lambda i: (i, 0)),
                   pl.BlockSpec((NUM_GRAPHS, D_NODE), lambda i: (0, 0))),
        out_shape=(jax.ShapeDtypeStruct((N_NODES, NUM_GRAPHS), F32),
                   jax.ShapeDtypeStruct((NUM_GRAPHS, D_NODE), F32)),
    )(batch2d)


def _edge_layer(xr, xc, e, wer, wec, wee, be, wn1x, wn1e, bn1, store_e):
    d_in = e.shape[1]
    e_out = wee.shape[1]
    nblk = N_EDGES // BLK_E

    def body(xr_ref, xc_ref, e_ref, wer_ref, wec_ref, wee_ref, be_ref,
             wn1x_ref, wn1e_ref, bn1_ref, *out_refs):
        xr_b = xr_ref[...].astype(BF16)
        xc_b = xc_ref[...].astype(BF16)
        e_b = e_ref[...].astype(BF16)
        acc = jnp.dot(xr_b, wer_ref[...].astype(BF16),
                      preferred_element_type=F32)
        acc += jnp.dot(xc_b, wec_ref[...].astype(BF16),
                       preferred_element_type=F32)
        acc += jnp.dot(e_b, wee_ref[...].astype(BF16),
                       preferred_element_type=F32)
        enew = jnp.maximum(acc + be_ref[...], 0.0)
        m = jnp.dot(xr_b, wn1x_ref[...].astype(BF16),
                    preferred_element_type=F32)
        m += jnp.dot(enew.astype(BF16), wn1e_ref[...].astype(BF16),
                     preferred_element_type=F32)
        m = jnp.maximum(m + bn1_ref[...], 0.0)
        if store_e:
            out_refs[0][...] = enew.astype(BF16)
            out_refs[1][...] = m
        else:
            out_refs[0][...] = m

    full = lambda shape: pl.BlockSpec(shape, lambda i: (0, 0))
    out_specs = [pl.BlockSpec((BLK_E, e_out), lambda i: (i, 0)),
                 pl.BlockSpec((BLK_E, D_NODE), lambda i: (i, 0))]
    out_shape = [jax.ShapeDtypeStruct((N_EDGES, e_out), BF16),
                 jax.ShapeDtypeStruct((N_EDGES, D_NODE), F32)]
    if not store_e:
        out_specs, out_shape = out_specs[1:], out_shape[1:]

    res = pl.pallas_call(
        body,
        grid=(nblk,),
        in_specs=[
            pl.BlockSpec((BLK_E, D_NODE), lambda i: (i, 0)),
            pl.BlockSpec((BLK_E, D_NODE), lambda i: (i, 0)),
            pl.BlockSpec((BLK_E, d_in), lambda i: (i, 0)),
            full((D_NODE, e_out)),
            full((D_NODE, e_out)),
            full((d_in, e_out)),
            full((1, e_out)),
            full((D_NODE, D_NODE)),
            full((e_out, D_NODE)),
            full((1, D_NODE)),
        ],
        out_specs=tuple(out_specs),
        out_shape=tuple(out_shape),
    )(xr, xc, e, wer, wec, wee, be.reshape(1, -1), wn1x, wn1e,
      bn1.reshape(1, -1))
    return res if store_e else (None, res[0])


def _node_layer(x, msum, cnts, ot, w2x, w2a, b2):
    nblk = N_NODES // BLK_N

    def body(x_ref, ms_ref, cnt_ref, ot_ref,
             w2x_ref, w2a_ref, b2_ref, xn_ref, ps_ref):
        i = pl.program_id(0)
        ms = ms_ref[0] + ms_ref[1]
        cnt = cnt_ref[0] + cnt_ref[1]
        inv = 1.0 / jnp.maximum(cnt[:, 0:1], 1.0)
        agg = ms * inv
        xn = jnp.dot(x_ref[...].astype(BF16), w2x_ref[...].astype(BF16),
                     preferred_element_type=F32)
        xn += jnp.dot(agg.astype(BF16), w2a_ref[...].astype(BF16),
                      preferred_element_type=F32)
        xn += b2_ref[...]
        xn_ref[...] = xn
        pp = lax.dot_general(ot_ref[...], xn, (((0,), (0,)), ((), ())),
                             preferred_element_type=F32)

        @pl.when(i == 0)
        def _():
            ps_ref[...] = pp

        @pl.when(i > 0)
        def _():
            ps_ref[...] += pp

    full = lambda shape: pl.BlockSpec(shape, lambda i: (0, 0))
    return pl.pallas_call(
        body,
        grid=(nblk,),
        in_specs=[
            pl.BlockSpec((BLK_N, D_NODE), lambda i: (i, 0)),
            pl.BlockSpec((NC, BLK_N, D_NODE), lambda i: (0, i, 0)),
            pl.BlockSpec((NC, BLK_N, D_NODE), lambda i: (0, i, 0)),
            pl.BlockSpec((BLK_N, NUM_GRAPHS), lambda i: (i, 0)),
            full((D_NODE, D_NODE)),
            full((D_NODE, D_NODE)),
            full((1, D_NODE)),
        ],
        out_specs=(pl.BlockSpec((BLK_N, D_NODE), lambda i: (i, 0)),
                   pl.BlockSpec((NUM_GRAPHS, D_NODE), lambda i: (0, 0))),
        out_shape=(jax.ShapeDtypeStruct((N_NODES, D_NODE), F32),
                   jax.ShapeDtypeStruct((NUM_GRAPHS, D_NODE), F32)),
    )(x, msum, cnts, ot, w2x, w2a, b2.reshape(1, -1))


def _glob_layer(psum, gcnt, u, wg_u, wg_p, bg):
    g_out = wg_p.shape[1]
    has_u = u is not None

    def body(*refs):
        if has_u:
            ps_ref, g_ref, u_ref, wgu_ref, wgp_ref, bg_ref, o_ref = refs
        else:
            ps_ref, g_ref, wgp_ref, bg_ref, o_ref = refs
        pooled = ps_ref[...] / jnp.maximum(g_ref[...], 1.0)
        acc = jnp.dot(pooled, wgp_ref[...], preferred_element_type=F32)
        if has_u:
            acc += jnp.dot(u_ref[...], wgu_ref[...], preferred_element_type=F32)
        o_ref[...] = acc + bg_ref[...]

    args = (psum, gcnt) + ((u, wg_u) if has_u else ()) + (wg_p, bg.reshape(1, -1))
    return pl.pallas_call(
        body, out_shape=jax.ShapeDtypeStruct((NUM_GRAPHS, g_out), F32),
    )(*args)


def _head(u, w1, b1, w2, b2):
    def body(u_ref, w1_ref, b1_ref, w2_ref, b2_ref, o_ref):
        y = jnp.dot(u_ref[...], w1_ref[...], preferred_element_type=F32)
        y += b1_ref[...]
        y = jnp.where(y > 0, y, jnp.exp(y) - 1.0)
        o_ref[...] = jnp.dot(y, w2_ref[...],
                             preferred_element_type=F32) + b2_ref[...]

    return pl.pallas_call(
        body, out_shape=jax.ShapeDtypeStruct((NUM_GRAPHS, w2.shape[1]), F32),
    )(u, w1, b1.reshape(1, -1), w2, b2.reshape(1, -1))


def kernel(x, edge_index, edge_attr, batch, params):
    row = edge_index[0].astype(jnp.int32)
    col = edge_index[1].astype(jnp.int32)
    batch2d = batch.astype(jnp.int32).reshape(N_NODES, 1)

    zrows = jnp.zeros((ROWS_PER_TILE, D_NODE), F32)

    xcur = _bn_x(x, params["bn_node"]["gamma"], params["bn_node"]["beta"])

    s128, q128 = _bn_e_stats(edge_attr.reshape(N_EDGES // 8, 128))
    s16 = s128.reshape(8, 16).sum(0)
    q16 = q128.reshape(8, 16).sum(0)
    mu_e = s16 / N_EDGES
    var_e = q16 / N_EDGES - mu_e * mu_e
    scale_e = params["bn_edge"]["gamma"] * lax.rsqrt(var_e + 1e-5)
    shift_e = params["bn_edge"]["beta"] - mu_e * scale_e

    cnts = _sc_segment_sum(jnp.ones((N_EDGES, D_NODE), F32), col, zrows)
    ot, gcnt = _pool_precompute(batch2d)

    ecur = edge_attr
    u = None
    for li in range(6):
        p = params["meta%d" % (li + 1)]
        we = p["edge"]["w"]
        be = p["edge"]["b"]
        wer, wec, wee = we[:D_NODE], we[D_NODE:2 * D_NODE], we[2 * D_NODE:]
        if li == 0:
            wee = scale_e[:, None] * wee
            be = be + shift_e @ we[2 * D_NODE:]
        wn1 = p["node1"]["w"]
        wn1x, wn1e = wn1[:D_NODE], wn1[D_NODE:]
        xr, xc = _sc_gather2(xcur, row, col)
        enew, m = _edge_layer(xr, xc, ecur, wer, wec, wee, be,
                              wn1x, wn1e, p["node1"]["b"], store_e=(li < 5))
        msum = _sc_segment_sum(m, col, zrows)
        wn2 = p["node2"]["w"]
        xcur, psum = _node_layer(xcur, msum, cnts, ot,
                                 wn2[:D_NODE], wn2[D_NODE:], p["node2"]["b"])
        wg = p["glob"]["w"]
        if u is None:
            u = _glob_layer(psum, gcnt, None, None, wg, p["glob"]["b"])
        else:
            u_in = wg.shape[0] - D_NODE
            u = _glob_layer(psum, gcnt, u, wg[:u_in], wg[u_in:],
                            p["glob"]["b"])
        ecur = enew

    out2 = _head(u, params["lin1"]["w"], params["lin1"]["b"],
                 params["lin2"]["w"], params["lin2"]["b"])
    return (u, out2)


# speedup vs baseline: 2.7776x; 1.0148x over previous
import functools

import jax
import jax.numpy as jnp
from jax import lax
from jax.experimental import pallas as pl
from jax.experimental.pallas import tpu as pltpu
from jax.experimental.pallas import tpu_sc as plsc

F32 = jnp.float32
BF16 = jnp.bfloat16

N_NODES = 10000
N_EDGES = 160000
NUM_GRAPHS = 64
D_NODE = 128

NC, NS = 2, 16
NW = NC * NS
CHUNK = 128
N_CHUNKS = N_EDGES // CHUNK
CHUNKS_PER_SC = N_CHUNKS // NC
N_PAD = 10112
ROWS_PER_TILE = N_PAD // NS

BLK_E = 2000
BLK_N = 2000

_SC_MESH = dict(core_axis_name="c", subcore_axis_name="s")


def _sc_gather2(x, eidx):
    G = 3
    mesh = plsc.VectorSubcoreMesh(**_SC_MESH)

    @functools.partial(
        pl.kernel,
        out_type=(jax.ShapeDtypeStruct((N_EDGES, D_NODE), F32),
                  jax.ShapeDtypeStruct((N_EDGES, D_NODE), F32)),
        mesh=mesh,
        scratch_types=[
            pltpu.VMEM((G * CHUNK,), jnp.int32),
            pltpu.VMEM((G * CHUNK, D_NODE), F32),
            pltpu.VMEM((G * CHUNK,), jnp.int32),
            pltpu.VMEM((G * CHUNK, D_NODE), F32),
            pltpu.SemaphoreType.DMA,
            pltpu.SemaphoreType.DMA,
        ],
    )
    def k(x_hbm, eidx_hbm, xr_hbm, xc_hbm, ridx, rbuf, cidx, cbuf,
          sem_r, sem_c):
        wid = lax.axis_index("s") * NC + lax.axis_index("c")
        start = 39 * wid + jnp.minimum(wid, N_CHUNKS % NW)

        def group(i, carry):
            base = (start + i * G) * CHUNK
            pltpu.sync_copy(eidx_hbm.at[pl.ds(base, G * CHUNK)], ridx)
            pltpu.sync_copy(eidx_hbm.at[pl.ds(N_EDGES + base, G * CHUNK)],
                            cidx)
            ds = []
            for j in range(G):
                sl = pl.ds(j * CHUNK, CHUNK)
                ds.append(pltpu.async_copy(
                    x_hbm.at[ridx.at[sl]], rbuf.at[sl], sem_r))
                ds.append(pltpu.async_copy(
                    x_hbm.at[cidx.at[sl]], cbuf.at[sl], sem_c))
            for d in ds:
                d.wait()
            pltpu.sync_copy(rbuf, xr_hbm.at[pl.ds(base, G * CHUNK)])
            pltpu.sync_copy(cbuf, xc_hbm.at[pl.ds(base, G * CHUNK)])
            return carry

        lax.fori_loop(0, 39 // G, group, 0)

        @pl.when(wid < N_CHUNKS % NW)
        def _():
            base = (start + 39) * CHUNK
            one = pl.ds(0, CHUNK)
            pltpu.sync_copy(eidx_hbm.at[pl.ds(base, CHUNK)], ridx.at[one])
            pltpu.sync_copy(eidx_hbm.at[pl.ds(N_EDGES + base, CHUNK)],
                            cidx.at[one])
            dr = pltpu.async_copy(x_hbm.at[ridx.at[one]], rbuf.at[one], sem_r)
            dc = pltpu.async_copy(x_hbm.at[cidx.at[one]], cbuf.at[one], sem_c)
            dr.wait()
            dc.wait()
            pltpu.sync_copy(rbuf.at[one], xr_hbm.at[pl.ds(base, CHUNK)])
            pltpu.sync_copy(cbuf.at[one], xc_hbm.at[pl.ds(base, CHUNK)])

    return k(x, eidx)


def _sc_segment_sum(m, eidx, zrows):
    G = 3
    mesh = plsc.VectorSubcoreMesh(**_SC_MESH)

    @functools.partial(
        pl.kernel,
        out_type=jax.ShapeDtypeStruct((NC, N_PAD, D_NODE), F32),
        mesh=mesh,
        scratch_types=[
            pltpu.VMEM((CHUNK,), jnp.int32),
            pltpu.VMEM((CHUNK,), jnp.int32),
            pltpu.VMEM((CHUNK,), jnp.int32),
            pltpu.VMEM((G * CHUNK, D_NODE), F32),
            pltpu.VMEM_SHARED((N_PAD, D_NODE), F32),
            pltpu.SemaphoreType.DMA,
            pltpu.SemaphoreType.DMA,
        ],
    )
    def k(m_hbm, eidx_hbm, z_hbm, out, idx0, idx1, idx2, buf, acc, sem_i,
          sem_s):
        cid = lax.axis_index("c")
        sid = lax.axis_index("s")
        my_rows = acc.at[pl.ds(sid * ROWS_PER_TILE, ROWS_PER_TILE)]
        pltpu.sync_copy(z_hbm, my_rows)
        plsc.subcore_barrier()

        idxs = [idx0, idx1, idx2]
        start = cid * CHUNKS_PER_SC + 39 * sid + jnp.minimum(
            sid, CHUNKS_PER_SC % NS)

        def group(i, carry):
            c0 = start + i * G
            base = c0 * CHUNK
            dm = pltpu.async_copy(m_hbm.at[pl.ds(base, G * CHUNK)], buf,
                                  sem_s)
            di = [pltpu.async_copy(
                eidx_hbm.at[pl.ds(N_EDGES + base + j * CHUNK, CHUNK)],
                idxs[j], sem_i)
                for j in range(G)]
            dm.wait()
            for d in di:
                d.wait()
            ds = [pltpu.async_copy(
                buf.at[pl.ds(j * CHUNK, CHUNK)], acc.at[idxs[j]], sem_s,
                add=True) for j in range(G)]
            for d in ds:
                d.wait()
            return carry

        lax.fori_loop(0, 39 // G, group, 0)

        @pl.when(sid < CHUNKS_PER_SC % NS)
        def _():
            base = (start + 39) * CHUNK
            pltpu.sync_copy(eidx_hbm.at[pl.ds(N_EDGES + base, CHUNK)], idx0)
            pltpu.sync_copy(m_hbm.at[pl.ds(base, CHUNK)],
                            buf.at[pl.ds(0, CHUNK)])
            pltpu.sync_copy(buf.at[pl.ds(0, CHUNK)], acc.at[idx0], add=True)

        plsc.subcore_barrier()
        pltpu.sync_copy(
            my_rows, out.at[cid, pl.ds(sid * ROWS_PER_TILE, ROWS_PER_TILE)])

    return k(m, eidx, zrows)


def _bn_x(x, gamma, beta):
    def body(x_ref, g_ref, b_ref, o_ref):
        xx = x_ref[...]
        mu = jnp.mean(xx, axis=0, keepdims=True)
        var = jnp.mean(xx * xx, axis=0, keepdims=True) - mu * mu
        o_ref[...] = (xx - mu) * lax.rsqrt(var + 1e-5) * g_ref[...] + b_ref[...]

    return pl.pallas_call(
        body, out_shape=jax.ShapeDtypeStruct(x.shape, F32),
    )(x, gamma.reshape(1, -1), beta.reshape(1, -1))


def _bn_e_stats(e):
    blk = N_EDGES // 8

    def body(e_ref, s_ref, q_ref):
        i = pl.program_id(0)
        ee = e_ref[...]
        s = jnp.sum(ee, axis=0, keepdims=True)
        q = jnp.sum(ee * ee, axis=0, keepdims=True)

        @pl.when(i == 0)
        def _():
            s_ref[...] = s
            q_ref[...] = q

        @pl.when(i > 0)
        def _():
            s_ref[...] += s
            q_ref[...] += q

    return pl.pallas_call(
        body,
        grid=(8,),
        in_specs=[pl.BlockSpec((blk, 16), lambda i: (i, 0))],
        out_specs=(pl.BlockSpec((1, 16), lambda i: (0, 0)),
                   pl.BlockSpec((1, 16), lambda i: (0, 0))),
        out_shape=(jax.ShapeDtypeStruct((1, 16), F32),
                   jax.ShapeDtypeStruct((1, 16), F32)),
    )(e)


def _pool_precompute(batch2d):
    nblk = N_NODES // BLK_N

    def body(b_ref, ot_ref, g_ref):
        i = pl.program_id(0)
        b = b_ref[...]
        gid = lax.broadcasted_iota(jnp.int32, (BLK_N, NUM_GRAPHS), 1)
        ot = (gid == b).astype(F32)
        ot_ref[...] = ot
        ones = jnp.ones((BLK_N, D_NODE), F32)
        gs = lax.dot_general(ot, ones, (((0,), (0,)), ((), ())),
                             preferred_element_type=F32)

        @pl.when(i == 0)
        def _():
            g_ref[...] = gs

        @pl.when(i > 0)
        def _():
            g_ref[...] += gs

    return pl.pallas_call(
        body,
        grid=(nblk,),
        in_specs=[pl.BlockSpec((BLK_N, 1), lambda i: (i, 0))],
        out_specs=(pl.BlockSpec((BLK_N, NUM_GRAPHS), lambda i: (i, 0)),
                   pl.BlockSpec((NUM_GRAPHS, D_NODE), lambda i: (0, 0))),
        out_shape=(jax.ShapeDtypeStruct((N_NODES, NUM_GRAPHS), F32),
                   jax.ShapeDtypeStruct((NUM_GRAPHS, D_NODE), F32)),
    )(batch2d)


def _edge_layer(xr, xc, e, wer, wec, wee, be, wn1x, wn1e, bn1, store_e):
    d_in = e.shape[1]
    e_out = wee.shape[1]
    nblk = N_EDGES // BLK_E

    def body(xr_ref, xc_ref, e_ref, wer_ref, wec_ref, wee_ref, be_ref,
             wn1x_ref, wn1e_ref, bn1_ref, *out_refs):
        xr_b = xr_ref[...].astype(BF16)
        xc_b = xc_ref[...].astype(BF16)
        e_b = e_ref[...].astype(BF16)
        acc = jnp.dot(xr_b, wer_ref[...].astype(BF16),
                      preferred_element_type=F32)
        acc += jnp.dot(xc_b, wec_ref[...].astype(BF16),
                       preferred_element_type=F32)
        acc += jnp.dot(e_b, wee_ref[...].astype(BF16),
                       preferred_element_type=F32)
        enew = jnp.maximum(acc + be_ref[...], 0.0)
        m = jnp.dot(xr_b, wn1x_ref[...].astype(BF16),
                    preferred_element_type=F32)
        m += jnp.dot(enew.astype(BF16), wn1e_ref[...].astype(BF16),
                     preferred_element_type=F32)
        m = jnp.maximum(m + bn1_ref[...], 0.0)
        if store_e:
            out_refs[0][...] = enew.astype(BF16)
            out_refs[1][...] = m
        else:
            out_refs[0][...] = m

    full = lambda shape: pl.BlockSpec(shape, lambda i: (0, 0))
    out_specs = [pl.BlockSpec((BLK_E, e_out), lambda i: (i, 0)),
                 pl.BlockSpec((BLK_E, D_NODE), lambda i: (i, 0))]
    out_shape = [jax.ShapeDtypeStruct((N_EDGES, e_out), BF16),
                 jax.ShapeDtypeStruct((N_EDGES, D_NODE), F32)]
    if not store_e:
        out_specs, out_shape = out_specs[1:], out_shape[1:]

    res = pl.pallas_call(
        body,
        grid=(nblk,),
        in_specs=[
            pl.BlockSpec((BLK_E, D_NODE), lambda i: (i, 0)),
            pl.BlockSpec((BLK_E, D_NODE), lambda i: (i, 0)),
            pl.BlockSpec((BLK_E, d_in), lambda i: (i, 0)),
            full((D_NODE, e_out)),
            full((D_NODE, e_out)),
            full((d_in, e_out)),
            full((1, e_out)),
            full((D_NODE, D_NODE)),
            full((e_out, D_NODE)),
            full((1, D_NODE)),
        ],
        out_specs=tuple(out_specs),
        out_shape=tuple(out_shape),
    )(xr, xc, e, wer, wec, wee, be.reshape(1, -1), wn1x, wn1e,
      bn1.reshape(1, -1))
    return res if store_e else (None, res[0])


def _node_layer(x, msum, cnts, ot, w2x, w2a, b2):
    nblk = N_NODES // BLK_N

    def body(x_ref, ms_ref, cnt_ref, ot_ref,
             w2x_ref, w2a_ref, b2_ref, xn_ref, ps_ref):
        i = pl.program_id(0)
        ms = ms_ref[0] + ms_ref[1]
        cnt = cnt_ref[0] + cnt_ref[1]
        inv = 1.0 / jnp.maximum(cnt[:, 0:1], 1.0)
        agg = ms * inv
        xn = jnp.dot(x_ref[...].astype(BF16), w2x_ref[...].astype(BF16),
                     preferred_element_type=F32)
        xn += jnp.dot(agg.astype(BF16), w2a_ref[...].astype(BF16),
                      preferred_element_type=F32)
        xn += b2_ref[...]
        xn_ref[...] = xn
        pp = lax.dot_general(ot_ref[...], xn, (((0,), (0,)), ((), ())),
                             preferred_element_type=F32)

        @pl.when(i == 0)
        def _():
            ps_ref[...] = pp

        @pl.when(i > 0)
        def _():
            ps_ref[...] += pp

    full = lambda shape: pl.BlockSpec(shape, lambda i: (0, 0))
    return pl.pallas_call(
        body,
        grid=(nblk,),
        in_specs=[
            pl.BlockSpec((BLK_N, D_NODE), lambda i: (i, 0)),
            pl.BlockSpec((NC, BLK_N, D_NODE), lambda i: (0, i, 0)),
            pl.BlockSpec((NC, BLK_N, D_NODE), lambda i: (0, i, 0)),
            pl.BlockSpec((BLK_N, NUM_GRAPHS), lambda i: (i, 0)),
            full((D_NODE, D_NODE)),
            full((D_NODE, D_NODE)),
            full((1, D_NODE)),
        ],
        out_specs=(pl.BlockSpec((BLK_N, D_NODE), lambda i: (i, 0)),
                   pl.BlockSpec((NUM_GRAPHS, D_NODE), lambda i: (0, 0))),
        out_shape=(jax.ShapeDtypeStruct((N_NODES, D_NODE), F32),
                   jax.ShapeDtypeStruct((NUM_GRAPHS, D_NODE), F32)),
    )(x, msum, cnts, ot, w2x, w2a, b2.reshape(1, -1))


def _glob_layer(psum, gcnt, u, wg_u, wg_p, bg):
    g_out = wg_p.shape[1]
    has_u = u is not None

    def body(*refs):
        if has_u:
            ps_ref, g_ref, u_ref, wgu_ref, wgp_ref, bg_ref, o_ref = refs
        else:
            ps_ref, g_ref, wgp_ref, bg_ref, o_ref = refs
        pooled = ps_ref[...] / jnp.maximum(g_ref[...], 1.0)
        acc = jnp.dot(pooled, wgp_ref[...], preferred_element_type=F32)
        if has_u:
            acc += jnp.dot(u_ref[...], wgu_ref[...], preferred_element_type=F32)
        o_ref[...] = acc + bg_ref[...]

    args = (psum, gcnt) + ((u, wg_u) if has_u else ()) + (wg_p, bg.reshape(1, -1))
    return pl.pallas_call(
        body, out_shape=jax.ShapeDtypeStruct((NUM_GRAPHS, g_out), F32),
    )(*args)


def _head(u, w1, b1, w2, b2):
    def body(u_ref, w1_ref, b1_ref, w2_ref, b2_ref, o_ref):
        y = jnp.dot(u_ref[...], w1_ref[...], preferred_element_type=F32)
        y += b1_ref[...]
        y = jnp.where(y > 0, y, jnp.exp(y) - 1.0)
        o_ref[...] = jnp.dot(y, w2_ref[...],
                             preferred_element_type=F32) + b2_ref[...]

    return pl.pallas_call(
        body, out_shape=jax.ShapeDtypeStruct((NUM_GRAPHS, w2.shape[1]), F32),
    )(u, w1, b1.reshape(1, -1), w2, b2.reshape(1, -1))


def kernel(x, edge_index, edge_attr, batch, params):
    eidx = edge_index.astype(jnp.int32).reshape(-1)
    batch2d = batch.astype(jnp.int32).reshape(N_NODES, 1)

    zrows = jnp.zeros((ROWS_PER_TILE, D_NODE), F32)

    xcur = _bn_x(x, params["bn_node"]["gamma"], params["bn_node"]["beta"])

    s16, q16 = _bn_e_stats(edge_attr)
    mu_e = s16.reshape(-1) / N_EDGES
    var_e = q16.reshape(-1) / N_EDGES - mu_e * mu_e
    scale_e = params["bn_edge"]["gamma"] * lax.rsqrt(var_e + 1e-5)
    shift_e = params["bn_edge"]["beta"] - mu_e * scale_e

    cnts = _sc_segment_sum(jnp.ones((N_EDGES, D_NODE), F32), eidx, zrows)
    ot, gcnt = _pool_precompute(batch2d)

    ecur = edge_attr
    u = None
    for li in range(6):
        p = params["meta%d" % (li + 1)]
        we = p["edge"]["w"]
        be = p["edge"]["b"]
        wer, wec, wee = we[:D_NODE], we[D_NODE:2 * D_NODE], we[2 * D_NODE:]
        if li == 0:
            wee = scale_e[:, None] * wee
            be = be + shift_e @ we[2 * D_NODE:]
        wn1 = p["node1"]["w"]
        wn1x, wn1e = wn1[:D_NODE], wn1[D_NODE:]
        xr, xc = _sc_gather2(xcur, eidx)
        enew, m = _edge_layer(xr, xc, ecur, wer, wec, wee, be,
                              wn1x, wn1e, p["node1"]["b"], store_e=(li < 5))
        msum = _sc_segment_sum(m, eidx, zrows)
        wn2 = p["node2"]["w"]
        xcur, psum = _node_layer(xcur, msum, cnts, ot,
                                 wn2[:D_NODE], wn2[D_NODE:], p["node2"]["b"])
        wg = p["glob"]["w"]
        if u is None:
            u = _glob_layer(psum, gcnt, None, None, wg, p["glob"]["b"])
        else:
            u_in = wg.shape[0] - D_NODE
            u = _glob_layer(psum, gcnt, u, wg[:u_in], wg[u_in:],
                            p["glob"]["b"])
        ecur = enew

    out2 = _head(u, params["lin1"]["w"], params["lin1"]["b"],
                 params["lin2"]["w"], params["lin2"]["b"])
    return (u, out2)


# speedup vs baseline: 2.8876x; 1.0396x over previous
import functools

import jax
import jax.numpy as jnp
from jax import lax
from jax.experimental import pallas as pl
from jax.experimental.pallas import tpu as pltpu
from jax.experimental.pallas import tpu_sc as plsc

F32 = jnp.float32
BF16 = jnp.bfloat16

N_NODES = 10000
N_EDGES = 160000
NUM_GRAPHS = 64
D_NODE = 128

NC, NS = 2, 16
NW = NC * NS
CHUNK = 128
N_CHUNKS = N_EDGES // CHUNK
CHUNKS_PER_SC = N_CHUNKS // NC
N_PAD = 10112
ROWS_PER_TILE = N_PAD // NS

BLK_E = 4000
BLK_N = 2000

_SC_MESH = dict(core_axis_name="c", subcore_axis_name="s")


def _sc_gather2(x, eidx):
    G = 3
    mesh = plsc.VectorSubcoreMesh(**_SC_MESH)

    @functools.partial(
        pl.kernel,
        out_type=(jax.ShapeDtypeStruct((N_EDGES, D_NODE), F32),
                  jax.ShapeDtypeStruct((N_EDGES, D_NODE), F32)),
        mesh=mesh,
        scratch_types=[
            pltpu.VMEM((G * CHUNK,), jnp.int32),
            pltpu.VMEM((G * CHUNK, D_NODE), F32),
            pltpu.VMEM((G * CHUNK,), jnp.int32),
            pltpu.VMEM((G * CHUNK, D_NODE), F32),
            pltpu.SemaphoreType.DMA,
            pltpu.SemaphoreType.DMA,
        ],
    )
    def k(x_hbm, eidx_hbm, xr_hbm, xc_hbm, ridx, rbuf, cidx, cbuf,
          sem_r, sem_c):
        wid = lax.axis_index("s") * NC + lax.axis_index("c")
        start = 39 * wid + jnp.minimum(wid, N_CHUNKS % NW)

        def group(i, carry):
            base = (start + i * G) * CHUNK
            pltpu.sync_copy(eidx_hbm.at[pl.ds(base, G * CHUNK)], ridx)
            pltpu.sync_copy(eidx_hbm.at[pl.ds(N_EDGES + base, G * CHUNK)],
                            cidx)
            ds = []
            for j in range(G):
                sl = pl.ds(j * CHUNK, CHUNK)
                ds.append(pltpu.async_copy(
                    x_hbm.at[ridx.at[sl]], rbuf.at[sl], sem_r))
                ds.append(pltpu.async_copy(
                    x_hbm.at[cidx.at[sl]], cbuf.at[sl], sem_c))
            for d in ds:
                d.wait()
            pltpu.sync_copy(rbuf, xr_hbm.at[pl.ds(base, G * CHUNK)])
            pltpu.sync_copy(cbuf, xc_hbm.at[pl.ds(base, G * CHUNK)])
            return carry

        lax.fori_loop(0, 39 // G, group, 0)

        @pl.when(wid < N_CHUNKS % NW)
        def _():
            base = (start + 39) * CHUNK
            one = pl.ds(0, CHUNK)
            pltpu.sync_copy(eidx_hbm.at[pl.ds(base, CHUNK)], ridx.at[one])
            pltpu.sync_copy(eidx_hbm.at[pl.ds(N_EDGES + base, CHUNK)],
                            cidx.at[one])
            dr = pltpu.async_copy(x_hbm.at[ridx.at[one]], rbuf.at[one], sem_r)
            dc = pltpu.async_copy(x_hbm.at[cidx.at[one]], cbuf.at[one], sem_c)
            dr.wait()
            dc.wait()
            pltpu.sync_copy(rbuf.at[one], xr_hbm.at[pl.ds(base, CHUNK)])
            pltpu.sync_copy(cbuf.at[one], xc_hbm.at[pl.ds(base, CHUNK)])

    return k(x, eidx)


def _sc_segment_sum(m, eidx, zrows):
    G = 3
    mesh = plsc.VectorSubcoreMesh(**_SC_MESH)

    @functools.partial(
        pl.kernel,
        out_type=jax.ShapeDtypeStruct((NC, N_PAD, D_NODE), F32),
        mesh=mesh,
        scratch_types=[
            pltpu.VMEM((CHUNK,), jnp.int32),
            pltpu.VMEM((CHUNK,), jnp.int32),
            pltpu.VMEM((CHUNK,), jnp.int32),
            pltpu.VMEM((G * CHUNK, D_NODE), F32),
            pltpu.VMEM_SHARED((N_PAD, D_NODE), F32),
            pltpu.SemaphoreType.DMA,
            pltpu.SemaphoreType.DMA,
        ],
    )
    def k(m_hbm, eidx_hbm, z_hbm, out, idx0, idx1, idx2, buf, acc, sem_i,
          sem_s):
        cid = lax.axis_index("c")
        sid = lax.axis_index("s")
        my_rows = acc.at[pl.ds(sid * ROWS_PER_TILE, ROWS_PER_TILE)]
        pltpu.sync_copy(z_hbm, my_rows)
        plsc.subcore_barrier()

        idxs = [idx0, idx1, idx2]
        start = cid * CHUNKS_PER_SC + 39 * sid + jnp.minimum(
            sid, CHUNKS_PER_SC % NS)

        def group(i, carry):
            c0 = start + i * G
            base = c0 * CHUNK
            dm = pltpu.async_copy(m_hbm.at[pl.ds(base, G * CHUNK)], buf,
                                  sem_s)
            di = [pltpu.async_copy(
                eidx_hbm.at[pl.ds(N_EDGES + base + j * CHUNK, CHUNK)],
                idxs[j], sem_i)
                for j in range(G)]
            dm.wait()
            for d in di:
                d.wait()
            ds = [pltpu.async_copy(
                buf.at[pl.ds(j * CHUNK, CHUNK)], acc.at[idxs[j]], sem_s,
                add=True) for j in range(G)]
            for d in ds:
                d.wait()
            return carry

        lax.fori_loop(0, 39 // G, group, 0)

        @pl.when(sid < CHUNKS_PER_SC % NS)
        def _():
            base = (start + 39) * CHUNK
            pltpu.sync_copy(eidx_hbm.at[pl.ds(N_EDGES + base, CHUNK)], idx0)
            pltpu.sync_copy(m_hbm.at[pl.ds(base, CHUNK)],
                            buf.at[pl.ds(0, CHUNK)])
            pltpu.sync_copy(buf.at[pl.ds(0, CHUNK)], acc.at[idx0], add=True)

        plsc.subcore_barrier()
        pltpu.sync_copy(
            my_rows, out.at[cid, pl.ds(sid * ROWS_PER_TILE, ROWS_PER_TILE)])

    return k(m, eidx, zrows)


def _bn_x(x, gamma, beta):
    def body(x_ref, g_ref, b_ref, o_ref):
        xx = x_ref[...]
        mu = jnp.mean(xx, axis=0, keepdims=True)
        var = jnp.mean(xx * xx, axis=0, keepdims=True) - mu * mu
        o_ref[...] = (xx - mu) * lax.rsqrt(var + 1e-5) * g_ref[...] + b_ref[...]

    return pl.pallas_call(
        body, out_shape=jax.ShapeDtypeStruct(x.shape, F32),
    )(x, gamma.reshape(1, -1), beta.reshape(1, -1))


def _bn_e_stats(e):
    blk = N_EDGES // 8

    def body(e_ref, s_ref, q_ref):
        i = pl.program_id(0)
        ee = e_ref[...]
        s = jnp.sum(ee, axis=0, keepdims=True)
        q = jnp.sum(ee * ee, axis=0, keepdims=True)

        @pl.when(i == 0)
        def _():
            s_ref[...] = s
            q_ref[...] = q

        @pl.when(i > 0)
        def _():
            s_ref[...] += s
            q_ref[...] += q

    return pl.pallas_call(
        body,
        grid=(8,),
        in_specs=[pl.BlockSpec((blk, 16), lambda i: (i, 0))],
        out_specs=(pl.BlockSpec((1, 16), lambda i: (0, 0)),
                   pl.BlockSpec((1, 16), lambda i: (0, 0))),
        out_shape=(jax.ShapeDtypeStruct((1, 16), F32),
                   jax.ShapeDtypeStruct((1, 16), F32)),
    )(e)


def _pool_precompute(batch2d):
    nblk = N_NODES // BLK_N

    def body(b_ref, ot_ref, g_ref):
        i = pl.program_id(0)
        b = b_ref[...]
        gid = lax.broadcasted_iota(jnp.int32, (BLK_N, NUM_GRAPHS), 1)
        ot = (gid == b).astype(F32)
        ot_ref[...] = ot
        ones = jnp.ones((BLK_N, D_NODE), F32)
        gs = lax.dot_general(ot, ones, (((0,), (0,)), ((), ())),
                             preferred_element_type=F32)

        @pl.when(i == 0)
        def _():
            g_ref[...] = gs

        @pl.when(i > 0)
        def _():
            g_ref[...] += gs

    return pl.pallas_call(
        body,
        grid=(nblk,),
        in_specs=[pl.BlockSpec((BLK_N, 1), lambda i: (i, 0))],
        out_specs=(pl.BlockSpec((BLK_N, NUM_GRAPHS), lambda i: (i, 0)),
                   pl.BlockSpec((NUM_GRAPHS, D_NODE), lambda i: (0, 0))),
        out_shape=(jax.ShapeDtypeStruct((N_NODES, NUM_GRAPHS), F32),
                   jax.ShapeDtypeStruct((NUM_GRAPHS, D_NODE), F32)),
    )(batch2d)


def _edge_layer(xr, xc, e, wer, wec, wee, be, wn1x, wn1e, bn1, store_e):
    d_in = e.shape[1]
    e_out = wee.shape[1]
    nblk = N_EDGES // BLK_E

    def body(xr_ref, xc_ref, e_ref, wer_ref, wec_ref, wee_ref, be_ref,
             wn1x_ref, wn1e_ref, bn1_ref, *out_refs):
        xr_b = xr_ref[...].astype(BF16)
        xc_b = xc_ref[...].astype(BF16)
        e_b = e_ref[...].astype(BF16)
        acc = jnp.dot(xr_b, wer_ref[...].astype(BF16),
                      preferred_element_type=F32)
        acc += jnp.dot(xc_b, wec_ref[...].astype(BF16),
                       preferred_element_type=F32)
        acc += jnp.dot(e_b, wee_ref[...].astype(BF16),
                       preferred_element_type=F32)
        enew = jnp.maximum(acc + be_ref[...], 0.0)
        m = jnp.dot(xr_b, wn1x_ref[...].astype(BF16),
                    preferred_element_type=F32)
        m += jnp.dot(enew.astype(BF16), wn1e_ref[...].astype(BF16),
                     preferred_element_type=F32)
        m = jnp.maximum(m + bn1_ref[...], 0.0)
        if store_e:
            out_refs[0][...] = enew.astype(BF16)
            out_refs[1][...] = m
        else:
            out_refs[0][...] = m

    full = lambda shape: pl.BlockSpec(shape, lambda i: (0, 0))
    out_specs = [pl.BlockSpec((BLK_E, e_out), lambda i: (i, 0)),
                 pl.BlockSpec((BLK_E, D_NODE), lambda i: (i, 0))]
    out_shape = [jax.ShapeDtypeStruct((N_EDGES, e_out), BF16),
                 jax.ShapeDtypeStruct((N_EDGES, D_NODE), F32)]
    if not store_e:
        out_specs, out_shape = out_specs[1:], out_shape[1:]

    res = pl.pallas_call(
        body,
        grid=(nblk,),
        in_specs=[
            pl.BlockSpec((BLK_E, D_NODE), lambda i: (i, 0)),
            pl.BlockSpec((BLK_E, D_NODE), lambda i: (i, 0)),
            pl.BlockSpec((BLK_E, d_in), lambda i: (i, 0)),
            full((D_NODE, e_out)),
            full((D_NODE, e_out)),
            full((d_in, e_out)),
            full((1, e_out)),
            full((D_NODE, D_NODE)),
            full((e_out, D_NODE)),
            full((1, D_NODE)),
        ],
        out_specs=tuple(out_specs),
        out_shape=tuple(out_shape),
    )(xr, xc, e, wer, wec, wee, be.reshape(1, -1), wn1x, wn1e,
      bn1.reshape(1, -1))
    return res if store_e else (None, res[0])


def _node_layer(x, msum, cnts, ot, w2x, w2a, b2):
    nblk = N_NODES // BLK_N

    def body(x_ref, ms_ref, cnt_ref, ot_ref,
             w2x_ref, w2a_ref, b2_ref, xn_ref, ps_ref):
        i = pl.program_id(0)
        ms = ms_ref[0] + ms_ref[1]
        cnt = cnt_ref[0] + cnt_ref[1]
        inv = 1.0 / jnp.maximum(cnt[:, 0:1], 1.0)
        agg = ms * inv
        xn = jnp.dot(x_ref[...].astype(BF16), w2x_ref[...].astype(BF16),
                     preferred_element_type=F32)
        xn += jnp.dot(agg.astype(BF16), w2a_ref[...].astype(BF16),
                      preferred_element_type=F32)
        xn += b2_ref[...]
        xn_ref[...] = xn
        pp = lax.dot_general(ot_ref[...], xn, (((0,), (0,)), ((), ())),
                             preferred_element_type=F32)

        @pl.when(i == 0)
        def _():
            ps_ref[...] = pp

        @pl.when(i > 0)
        def _():
            ps_ref[...] += pp

    full = lambda shape: pl.BlockSpec(shape, lambda i: (0, 0))
    return pl.pallas_call(
        body,
        grid=(nblk,),
        in_specs=[
            pl.BlockSpec((BLK_N, D_NODE), lambda i: (i, 0)),
            pl.BlockSpec((NC, BLK_N, D_NODE), lambda i: (0, i, 0)),
            pl.BlockSpec((NC, BLK_N, D_NODE), lambda i: (0, i, 0)),
            pl.BlockSpec((BLK_N, NUM_GRAPHS), lambda i: (i, 0)),
            full((D_NODE, D_NODE)),
            full((D_NODE, D_NODE)),
            full((1, D_NODE)),
        ],
        out_specs=(pl.BlockSpec((BLK_N, D_NODE), lambda i: (i, 0)),
                   pl.BlockSpec((NUM_GRAPHS, D_NODE), lambda i: (0, 0))),
        out_shape=(jax.ShapeDtypeStruct((N_NODES, D_NODE), F32),
                   jax.ShapeDtypeStruct((NUM_GRAPHS, D_NODE), F32)),
    )(x, msum, cnts, ot, w2x, w2a, b2.reshape(1, -1))


def _glob_layer(psum, gcnt, u, wg_u, wg_p, bg):
    g_out = wg_p.shape[1]
    has_u = u is not None

    def body(*refs):
        if has_u:
            ps_ref, g_ref, u_ref, wgu_ref, wgp_ref, bg_ref, o_ref = refs
        else:
            ps_ref, g_ref, wgp_ref, bg_ref, o_ref = refs
        pooled = ps_ref[...] / jnp.maximum(g_ref[...], 1.0)
        acc = jnp.dot(pooled, wgp_ref[...], preferred_element_type=F32)
        if has_u:
            acc += jnp.dot(u_ref[...], wgu_ref[...], preferred_element_type=F32)
        o_ref[...] = acc + bg_ref[...]

    args = (psum, gcnt) + ((u, wg_u) if has_u else ()) + (wg_p, bg.reshape(1, -1))
    return pl.pallas_call(
        body, out_shape=jax.ShapeDtypeStruct((NUM_GRAPHS, g_out), F32),
    )(*args)


def _head(u, w1, b1, w2, b2):
    def body(u_ref, w1_ref, b1_ref, w2_ref, b2_ref, o_ref):
        y = jnp.dot(u_ref[...], w1_ref[...], preferred_element_type=F32)
        y += b1_ref[...]
        y = jnp.where(y > 0, y, jnp.exp(y) - 1.0)
        o_ref[...] = jnp.dot(y, w2_ref[...],
                             preferred_element_type=F32) + b2_ref[...]

    return pl.pallas_call(
        body, out_shape=jax.ShapeDtypeStruct((NUM_GRAPHS, w2.shape[1]), F32),
    )(u, w1, b1.reshape(1, -1), w2, b2.reshape(1, -1))


def kernel(x, edge_index, edge_attr, batch, params):
    eidx = edge_index.astype(jnp.int32).reshape(-1)
    batch2d = batch.astype(jnp.int32).reshape(N_NODES, 1)

    zrows = jnp.zeros((ROWS_PER_TILE, D_NODE), F32)

    xcur = _bn_x(x, params["bn_node"]["gamma"], params["bn_node"]["beta"])

    s16, q16 = _bn_e_stats(edge_attr)
    mu_e = s16.reshape(-1) / N_EDGES
    var_e = q16.reshape(-1) / N_EDGES - mu_e * mu_e
    scale_e = params["bn_edge"]["gamma"] * lax.rsqrt(var_e + 1e-5)
    shift_e = params["bn_edge"]["beta"] - mu_e * scale_e

    cnts = _sc_segment_sum(jnp.ones((N_EDGES, D_NODE), F32), eidx, zrows)
    ot, gcnt = _pool_precompute(batch2d)

    ecur = edge_attr
    u = None
    for li in range(6):
        p = params["meta%d" % (li + 1)]
        we = p["edge"]["w"]
        be = p["edge"]["b"]
        wer, wec, wee = we[:D_NODE], we[D_NODE:2 * D_NODE], we[2 * D_NODE:]
        if li == 0:
            wee = scale_e[:, None] * wee
            be = be + shift_e @ we[2 * D_NODE:]
        wn1 = p["node1"]["w"]
        wn1x, wn1e = wn1[:D_NODE], wn1[D_NODE:]
        xr, xc = _sc_gather2(xcur, eidx)
        enew, m = _edge_layer(xr, xc, ecur, wer, wec, wee, be,
                              wn1x, wn1e, p["node1"]["b"], store_e=(li < 5))
        msum = _sc_segment_sum(m, eidx, zrows)
        wn2 = p["node2"]["w"]
        xcur, psum = _node_layer(xcur, msum, cnts, ot,
                                 wn2[:D_NODE], wn2[D_NODE:], p["node2"]["b"])
        wg = p["glob"]["w"]
        if u is None:
            u = _glob_layer(psum, gcnt, None, None, wg, p["glob"]["b"])
        else:
            u_in = wg.shape[0] - D_NODE
            u = _glob_layer(psum, gcnt, u, wg[:u_in], wg[u_in:],
                            p["glob"]["b"])
        ecur = enew

    out2 = _head(u, params["lin1"]["w"], params["lin1"]["b"],
                 params["lin2"]["w"], params["lin2"]["b"])
    return (u, out2)
